# baseline (speedup 1.0000x reference)
import functools

import jax
import jax.numpy as jnp
from jax import lax
from jax.experimental import pallas as pl
from jax.experimental.pallas import tpu as pltpu
from jax.experimental.pallas import tpu_sc as plsc

BATCH = 4096
HIST = 200
EMBED = 64
B = BATCH * HIST

NC = 2
NS = 16
NW = NC * NS

CHUNK = 128
B_PER_W = B // NW
NCH = B_PER_W // CHUNK
K = 4
NBLK = NCH // K
BLK_ROWS = K * CHUNK

_mesh = plsc.VectorSubcoreMesh(core_axis_name="c", subcore_axis_name="s")


@functools.partial(
    pl.kernel,
    mesh=_mesh,
    out_type=jax.ShapeDtypeStruct((B, EMBED), jnp.float32),
    scratch_types=[
        pltpu.VMEM((NCH, CHUNK), jnp.int32),
        pltpu.VMEM((2, BLK_ROWS, EMBED), jnp.float32),
        pltpu.SemaphoreType.DMA,
        pltpu.SemaphoreType.DMA,
    ],
    compiler_params=pltpu.CompilerParams(use_tc_tiling_on_sc=False),
)
def _emb(idx_hbm, table_hbm, out_hbm, idx_v, rows_v, gsem0, gsem1):
    wid = lax.axis_index("s") * NC + lax.axis_index("c")
    chunk0 = wid * NCH
    out0 = wid * B_PER_W

    pltpu.sync_copy(idx_hbm.at[pl.ds(chunk0, NCH)], idx_v)

    def fire_block(i, half, sem):
        for b in range(K):
            j = i * K + b
            pltpu.async_copy(
                table_hbm.at[idx_v.at[j]],
                rows_v.at[half, pl.ds(b * CHUNK, CHUNK)],
                sem,
            )

    def drain_block(half, sem):
        pltpu.make_async_copy(
            out_hbm.at[pl.ds(0, BLK_ROWS)], rows_v.at[half], sem
        ).wait()

    def out_block(i, half):
        pltpu.sync_copy(
            rows_v.at[half], out_hbm.at[pl.ds(out0 + i * BLK_ROWS, BLK_ROWS)]
        )

    fire_block(0, 0, gsem0)

    def body(t, carry):
        i0 = 2 * t
        i1 = i0 + 1
        fire_block(i1, 1, gsem1)
        drain_block(0, gsem0)
        out_block(i0, 0)

        @pl.when(i1 + 1 < NBLK)
        def _():
            fire_block(i1 + 1, 0, gsem0)

        drain_block(1, gsem1)
        out_block(i1, 1)
        return carry

    lax.fori_loop(0, NBLK // 2, body, 0)


def kernel(x, table):
    idx = x.reshape(B).astype(jnp.int32).reshape(B // CHUNK, CHUNK)
    out = _emb(idx, table)
    return out.reshape(BATCH, HIST, EMBED)


# speedup vs baseline: 1.0021x; 1.0021x over previous
import functools

import jax
import jax.numpy as jnp
from jax import lax
from jax.experimental import pallas as pl
from jax.experimental.pallas import tpu as pltpu
from jax.experimental.pallas import tpu_sc as plsc

BATCH = 4096
HIST = 200
EMBED = 64
B = BATCH * HIST

NC = 2
NS = 16
NW = NC * NS

CHUNK = 128
B_PER_W = B // NW
NCH = B_PER_W // CHUNK
K = 4
NBLK = NCH // K
BLK_ROWS = K * CHUNK

_mesh = plsc.VectorSubcoreMesh(core_axis_name="c", subcore_axis_name="s")


@functools.partial(
    pl.kernel,
    mesh=_mesh,
    out_type=jax.ShapeDtypeStruct((B, EMBED), jnp.float32),
    scratch_types=[
        pltpu.VMEM((B_PER_W,), jnp.int32),
        pltpu.VMEM((2, BLK_ROWS, EMBED), jnp.float32),
        pltpu.SemaphoreType.DMA,
        pltpu.SemaphoreType.DMA,
    ],
    compiler_params=pltpu.CompilerParams(use_tc_tiling_on_sc=False),
)
def _emb(idx_hbm, table_hbm, out_hbm, idx_v, rows_v, gsem0, gsem1):
    wid = lax.axis_index("s") * NC + lax.axis_index("c")
    out0 = wid * B_PER_W

    pltpu.sync_copy(idx_hbm.at[pl.ds(out0, B_PER_W)], idx_v)

    def fire_block(i, half, sem):
        for b in range(K):
            j = i * K + b
            pltpu.async_copy(
                table_hbm.at[idx_v.at[pl.ds(j * CHUNK, CHUNK)]],
                rows_v.at[half, pl.ds(b * CHUNK, CHUNK)],
                sem,
            )

    def drain_block(half, sem):
        pltpu.make_async_copy(
            out_hbm.at[pl.ds(0, BLK_ROWS)], rows_v.at[half], sem
        ).wait()

    def out_block(i, half):
        pltpu.sync_copy(
            rows_v.at[half], out_hbm.at[pl.ds(out0 + i * BLK_ROWS, BLK_ROWS)]
        )

    fire_block(0, 0, gsem0)

    def body(t, carry):
        i0 = 2 * t
        i1 = i0 + 1
        fire_block(i1, 1, gsem1)
        drain_block(0, gsem0)
        out_block(i0, 0)

        @pl.when(i1 + 1 < NBLK)
        def _():
            fire_block(i1 + 1, 0, gsem0)

        drain_block(1, gsem1)
        out_block(i1, 1)
        return carry

    lax.fori_loop(0, NBLK // 2, body, 0)


def kernel(x, table):
    idx = x.reshape(B).astype(jnp.int32)
    out = _emb(idx, table)
    return out.reshape(BATCH, HIST, EMBED)
